# Initial kernel scaffold; baseline (speedup 1.0000x reference)
import functools

import jax
import jax.numpy as jnp
from jax import lax
from jax.experimental import pallas as pl
from jax.experimental.pallas import tpu as pltpu
from jax.experimental.pallas import tpu_sc as plsc

N = 10000
E = 320000
D = 128
DO = 64
NC, NS = 2, 16
NW = NC * NS
EPW = E // NW
K = 80
NCH = EPW // K
RPT = N // NS
ZR = 125
R = 1000


def _sc_segment_partials(h, srcm, dstm):
    mesh = plsc.VectorSubcoreMesh(core_axis_name="c", subcore_axis_name="s")

    @functools.partial(
        pl.kernel,
        out_type=jax.ShapeDtypeStruct((NC, N, D), jnp.float32),
        mesh=mesh,
        scratch_types=[
            pltpu.VMEM((NCH, K), jnp.int32),
            pltpu.VMEM((NCH, K), jnp.int32),
            pltpu.VMEM((K, D), jnp.float32),
            pltpu.VMEM((ZR, D), jnp.float32),
            pltpu.VMEM_SHARED((N, D), jnp.float32),
            pltpu.SemaphoreType.DMA,
        ],
    )
    def k(h_hbm, src_hbm, dst_hbm, out_hbm, src_v, dst_v, rows_v, stage_v,
          acc, sem):
        c = lax.axis_index("c")
        s = lax.axis_index("s")
        wid = c * NS + s

        def zbody(i, carry):
            stage_v[i // 8, pl.ds((i % 8) * 16, 16)] = jnp.zeros(
                (16,), jnp.float32)
            return carry
        lax.fori_loop(0, ZR * (D // 16), zbody, 0)

        for j in range(RPT // ZR):
            pltpu.sync_copy(stage_v, acc.at[pl.ds(s * RPT + j * ZR, ZR)])
        plsc.subcore_barrier()

        pltpu.sync_copy(src_hbm.at[pl.ds(wid * NCH, NCH)], src_v)
        pltpu.sync_copy(dst_hbm.at[pl.ds(wid * NCH, NCH)], dst_v)

        def chunk(j, carry):
            pltpu.async_copy(h_hbm.at[src_v.at[j]], rows_v, sem).wait()
            pltpu.sync_copy(rows_v, acc.at[dst_v.at[j]], add=True)
            return carry
        lax.fori_loop(0, NCH, chunk, 0)
        plsc.subcore_barrier()

        for j in range(RPT // ZR):
            r0 = s * RPT + j * ZR
            pltpu.sync_copy(acc.at[pl.ds(r0, ZR)], stage_v)
            pltpu.sync_copy(stage_v, out_hbm.at[c, pl.ds(r0, ZR)])

    return k(h, srcm, dstm)


def _tc_layer(P, xin, W_rel, W_root, b_rel):
    def body(p_ref, x_ref, wr_ref, wt_ref, b_ref, o_ref):
        agg = p_ref[0] + p_ref[1]
        acc = jnp.dot(agg, wr_ref[...], preferred_element_type=jnp.float32)
        acc += jnp.dot(x_ref[...], wt_ref[...],
                       preferred_element_type=jnp.float32)
        o_ref[...] = jnp.tanh(acc + b_ref[...])

    return pl.pallas_call(
        body,
        grid=(N // R,),
        in_specs=[
            pl.BlockSpec((2, R, D), lambda i: (0, i, 0)),
            pl.BlockSpec((R, D), lambda i: (i, 0)),
            pl.BlockSpec((D, D), lambda i: (0, 0)),
            pl.BlockSpec((D, D), lambda i: (0, 0)),
            pl.BlockSpec((1, D), lambda i: (0, 0)),
        ],
        out_specs=pl.BlockSpec((R, D), lambda i: (i, 0)),
        out_shape=jax.ShapeDtypeStruct((N, D), jnp.float32),
    )(P, xin, W_rel, W_root, b_rel)


def _tc_final(Q, h, W_rel, W_root, b_rel, W_out, b_out):
    def body(q_ref, h_ref, wr_ref, wt_ref, br_ref, wo_ref, bo_ref, o_ref):
        agg = q_ref[0] + q_ref[1]
        acc = jnp.dot(agg, wr_ref[...], preferred_element_type=jnp.float32)
        acc += jnp.dot(h_ref[...], wt_ref[...],
                       preferred_element_type=jnp.float32)
        h2 = jnp.tanh(acc + br_ref[...])
        o_ref[...] = jnp.dot(h2, wo_ref[...],
                             preferred_element_type=jnp.float32) + bo_ref[...]

    return pl.pallas_call(
        body,
        grid=(N // R,),
        in_specs=[
            pl.BlockSpec((2, R, D), lambda i: (0, i, 0)),
            pl.BlockSpec((R, D), lambda i: (i, 0)),
            pl.BlockSpec((D, D), lambda i: (0, 0)),
            pl.BlockSpec((D, D), lambda i: (0, 0)),
            pl.BlockSpec((1, D), lambda i: (0, 0)),
            pl.BlockSpec((D, DO), lambda i: (0, 0)),
            pl.BlockSpec((1, DO), lambda i: (0, 0)),
        ],
        out_specs=pl.BlockSpec((R, DO), lambda i: (i, 0)),
        out_shape=jax.ShapeDtypeStruct((N, DO), jnp.float32),
    )(Q, h, W_rel, W_root, b_rel, W_out, b_out)


def kernel(x, edge_index, batch, W_rel0, b_rel0, W_root0, W_rel1, b_rel1,
           W_root1, W_out, b_out):
    srcm = edge_index[0].reshape(NW * NCH, K)
    dstm = edge_index[1].reshape(NW * NCH, K)
    P = _sc_segment_partials(x, srcm, dstm)
    h = _tc_layer(P, x, W_rel0, W_root0, b_rel0.reshape(1, D))
    Q = _sc_segment_partials(h, srcm, dstm)
    return _tc_final(Q, h, W_rel1, W_root1, b_rel1.reshape(1, D),
                     W_out, b_out.reshape(1, DO))


# speedup vs baseline: 15.2801x; 15.2801x over previous
import functools

import jax
import jax.numpy as jnp
from jax import lax
from jax.experimental import pallas as pl
from jax.experimental.pallas import tpu as pltpu
from jax.experimental.pallas import tpu_sc as plsc

N = 10000
NP = 10240
E = 320000
D = 128
DO = 64
NC, NS = 2, 16
NW = NC * NS
EPW = E // NW
K = 40
NCH = EPW // K
NBUF = 5
IB = 50
NBLK = NCH // IB
IBK = IB * K
RPT = NP // NS
R = 5000


def _sc_segment_partials(h, srcm, dstm):
    mesh = plsc.VectorSubcoreMesh(core_axis_name="c", subcore_axis_name="s")

    @functools.partial(
        pl.kernel,
        out_type=jax.ShapeDtypeStruct((NC, NP, D), jnp.float32),
        mesh=mesh,
        scratch_types=[
            pltpu.VMEM((2 * IBK,), jnp.int32),
            pltpu.VMEM((2 * IBK,), jnp.int32),
            pltpu.VMEM((NBUF * K, D), jnp.float32),
            pltpu.VMEM_SHARED((NP, D), jnp.float32),
        ] + [pltpu.SemaphoreType.DMA] * (NBUF + 2),
    )
    def k(h_hbm, src_hbm, dst_hbm, out_hbm, src_v, dst_v,
          rows_v, acc, *sems):
        c = lax.axis_index("c")
        s = lax.axis_index("s")
        wid = c * NS + s

        def gather(par, chunk, b):
            return pltpu.async_copy(
                h_hbm.at[src_v.at[pl.ds(par * IBK + chunk * K, K)]],
                rows_v.at[pl.ds(b * K, K)], sems[b])

        def wait_gather(par, chunk, b):
            pltpu.make_async_copy(
                h_hbm.at[src_v.at[pl.ds(par * IBK + chunk * K, K)]],
                rows_v.at[pl.ds(b * K, K)], sems[b]).wait()

        ebase = wid * EPW
        pltpu.sync_copy(src_hbm.at[pl.ds(ebase, IBK)],
                        src_v.at[pl.ds(0, IBK)])
        pltpu.sync_copy(dst_hbm.at[pl.ds(ebase, IBK)],
                        dst_v.at[pl.ds(0, IBK)])
        for b in range(NBUF - 1):
            gather(0, b, b)

        zbase = (NBUF - 1) * K
        def zbody(r, carry):
            for kk in range(D // 16):
                rows_v[zbase + r, pl.ds(kk * 16, 16)] = jnp.zeros(
                    (16,), jnp.float32)
            return carry
        lax.fori_loop(0, K, zbody, 0)
        zsem = sems[NBUF - 1]
        for j in range(RPT // K):
            pltpu.async_copy(rows_v.at[pl.ds(zbase, K)],
                             acc.at[pl.ds(s * RPT + j * K, K)], zsem)
        for j in range(RPT // K):
            pltpu.make_async_copy(rows_v.at[pl.ds(zbase, K)],
                                  acc.at[pl.ds(s * RPT + j * K, K)],
                                  zsem).wait()
        gather(0, NBUF - 1, NBUF - 1)
        plsc.subcore_barrier()

        def blk_body(blk, carry):
            par = lax.rem(blk, 2)
            nxt = 1 - par

            @pl.when(blk + 1 < NBLK)
            def _stage_next():
                pltpu.async_copy(
                    src_hbm.at[pl.ds(ebase + (blk + 1) * IBK, IBK)],
                    src_v.at[pl.ds(nxt * IBK, IBK)], sems[NBUF])
                pltpu.async_copy(
                    dst_hbm.at[pl.ds(ebase + (blk + 1) * IBK, IBK)],
                    dst_v.at[pl.ds(nxt * IBK, IBK)], sems[NBUF + 1])

            def group(gi, carry2):
                for b in range(NBUF):
                    chunk = gi * NBUF + b
                    wait_gather(par, chunk, b)
                    pltpu.sync_copy(rows_v.at[pl.ds(b * K, K)],
                                    acc.at[dst_v.at[pl.ds(par * IBK + chunk * K, K)]],
                        add=True)
                    gather(par, chunk + NBUF, b)
                return carry2
            lax.fori_loop(0, IB // NBUF - 1, group, 0)

            for b in range(NBUF):
                chunk = IB - NBUF + b
                wait_gather(par, chunk, b)
                pltpu.sync_copy(rows_v.at[pl.ds(b * K, K)],
                                acc.at[dst_v.at[pl.ds(par * IBK + chunk * K, K)]],
                        add=True)

                if b == 0:
                    @pl.when(blk + 1 < NBLK)
                    def _wait_stage():
                        pltpu.make_async_copy(
                            src_hbm.at[pl.ds(ebase + (blk + 1) * IBK, IBK)],
                            src_v.at[pl.ds(nxt * IBK, IBK)],
                            sems[NBUF]).wait()
                        pltpu.make_async_copy(
                            dst_hbm.at[pl.ds(ebase + (blk + 1) * IBK, IBK)],
                            dst_v.at[pl.ds(nxt * IBK, IBK)],
                            sems[NBUF + 1]).wait()

                @pl.when(blk + 1 < NBLK)
                def _head_next():
                    gather(nxt, b, b)
            return carry
        lax.fori_loop(0, NBLK, blk_body, 0)
        plsc.subcore_barrier()

        WR = 80
        for j in range(RPT // WR):
            r0 = s * RPT + j * WR
            half = (j % 2) * WR
            if j >= 2:
                pltpu.make_async_copy(
                    rows_v.at[pl.ds(half, WR)],
                    out_hbm.at[c, pl.ds(s * RPT + (j - 2) * WR, WR)],
                    sems[j % 2]).wait()
            pltpu.sync_copy(acc.at[pl.ds(r0, WR)], rows_v.at[pl.ds(half, WR)])
            pltpu.async_copy(rows_v.at[pl.ds(half, WR)],
                             out_hbm.at[c, pl.ds(r0, WR)], sems[j % 2])
        for j in (RPT // WR - 2, RPT // WR - 1):
            half = (j % 2) * WR
            pltpu.make_async_copy(
                rows_v.at[pl.ds(half, WR)],
                out_hbm.at[c, pl.ds(s * RPT + j * WR, WR)],
                sems[j % 2]).wait()

    return k(h, srcm, dstm)


def _tc_root(xin, W_root, b):
    def body(x_ref, w_ref, b_ref, o_ref):
        o_ref[...] = jnp.dot(x_ref[...], w_ref[...],
                             preferred_element_type=jnp.float32) + b_ref[...]

    return pl.pallas_call(
        body,
        grid=(N // R,),
        in_specs=[
            pl.BlockSpec((R, D), lambda i: (i, 0)),
            pl.BlockSpec((D, D), lambda i: (0, 0)),
            pl.BlockSpec((1, D), lambda i: (0, 0)),
        ],
        out_specs=pl.BlockSpec((R, D), lambda i: (i, 0)),
        out_shape=jax.ShapeDtypeStruct((N, D), jnp.float32),
    )(xin, W_root, b)


def _tc_layer(P, xr, W_rel):
    def body(p_ref, xr_ref, wr_ref, o_ref):
        agg = p_ref[0] + p_ref[1]
        acc = jnp.dot(agg, wr_ref[...], preferred_element_type=jnp.float32)
        o_ref[...] = jnp.tanh(acc + xr_ref[...])

    return pl.pallas_call(
        body,
        grid=(N // R,),
        in_specs=[
            pl.BlockSpec((2, R, D), lambda i: (0, i, 0)),
            pl.BlockSpec((R, D), lambda i: (i, 0)),
            pl.BlockSpec((D, D), lambda i: (0, 0)),
        ],
        out_specs=pl.BlockSpec((R, D), lambda i: (i, 0)),
        out_shape=jax.ShapeDtypeStruct((N, D), jnp.float32),
    )(P, xr, W_rel)


def _tc_final(Q, hr, W_rel, W_out, b_out):
    def body(q_ref, hr_ref, wr_ref, wo_ref, bo_ref, o_ref):
        agg = q_ref[0] + q_ref[1]
        acc = jnp.dot(agg, wr_ref[...], preferred_element_type=jnp.float32)
        h2 = jnp.tanh(acc + hr_ref[...])
        o_ref[...] = jnp.dot(h2, wo_ref[...],
                             preferred_element_type=jnp.float32) + bo_ref[...]

    return pl.pallas_call(
        body,
        grid=(N // R,),
        in_specs=[
            pl.BlockSpec((2, R, D), lambda i: (0, i, 0)),
            pl.BlockSpec((R, D), lambda i: (i, 0)),
            pl.BlockSpec((D, D), lambda i: (0, 0)),
            pl.BlockSpec((D, DO), lambda i: (0, 0)),
            pl.BlockSpec((1, DO), lambda i: (0, 0)),
        ],
        out_specs=pl.BlockSpec((R, DO), lambda i: (i, 0)),
        out_shape=jax.ShapeDtypeStruct((N, DO), jnp.float32),
    )(Q, hr, W_rel, W_out, b_out)


def kernel(x, edge_index, batch, W_rel0, b_rel0, W_root0, W_rel1, b_rel1,
           W_root1, W_out, b_out):
    srcm = edge_index[0]
    dstm = edge_index[1]
    xr = _tc_root(x, W_root0, b_rel0.reshape(1, D))
    P = _sc_segment_partials(x, srcm, dstm)
    h = _tc_layer(P, xr, W_rel0)
    hr = _tc_root(h, W_root1, b_rel1.reshape(1, D))
    Q = _sc_segment_partials(h, srcm, dstm)
    return _tc_final(Q, hr, W_rel1, W_out, b_out.reshape(1, DO))
